# baseline (speedup 1.0000x reference)
import math

import jax
import jax.numpy as jnp
from jax import lax
from jax.experimental import pallas as pl
from jax.experimental.pallas import tpu as pltpu

ROWS = 128
COLS = 100000
TEMP = 5.0
BC = 4096
NCB = math.ceil(COLS / BC)
_PADW = 100096
_ZLAST = _PADW - BC
_NG = ROWS // 8

_KS0 = 0
_KS1 = 42
_KS2 = 42 ^ 0x1BD11BDA
_ROT_A = (13, 15, 26, 6)
_ROT_B = (17, 29, 16, 24)


def _rounds(x0, x1, rots):
    for r in rots:
        x0 = x0 + x1
        x1 = (x1 << r) | lax.shift_right_logical(x1, 32 - r)
        x1 = x1 ^ x0
    return x0, x1


def _threefry_bits(e):
    x0 = jnp.zeros_like(e) + _KS0
    x1 = e + _KS1
    x0, x1 = _rounds(x0, x1, _ROT_A)
    x0, x1 = x0 + _KS1, x1 + (_KS2 + 1)
    x0, x1 = _rounds(x0, x1, _ROT_B)
    x0, x1 = x0 + _KS2, x1 + (_KS0 + 2)
    x0, x1 = _rounds(x0, x1, _ROT_A)
    x0, x1 = x0 + _KS0, x1 + (_KS1 + 3)
    x0, x1 = _rounds(x0, x1, _ROT_B)
    x0, x1 = x0 + _KS1, x1 + (_KS2 + 4)
    x0, x1 = _rounds(x0, x1, _ROT_A)
    x0, x1 = x0 + _KS2, x1 + (_KS0 + 5)
    return x0 ^ x1


def _zero_copy(cb, zbuf_ref, out_ref, zsem):
    off = pl.multiple_of(jnp.minimum(cb * BC, _ZLAST), 128)
    return pltpu.make_async_copy(zbuf_ref, out_ref.at[:, pl.ds(off, BC)], zsem)


def _fused_kernel(x_ref, out_ref, val_ref, idx_ref, zbuf_ref, patch_ref,
                  idxs_ref, zsem, ssem):
    cb = pl.program_id(0)

    @pl.when(cb == 0)
    def _():
        zbuf_ref[...] = jnp.zeros((ROWS, BC), jnp.float32)

    _zero_copy(cb, zbuf_ref, out_ref, zsem).start()

    jj = lax.broadcasted_iota(jnp.int32, (ROWS, BC), 1) + cb * BC
    ii = lax.broadcasted_iota(jnp.int32, (ROWS, BC), 0)
    e = ii * COLS + jj
    bits = _threefry_bits(e)
    mant = lax.shift_right_logical(bits, 9) | 0x3F800000
    u = lax.bitcast_convert_type(mant, jnp.float32) - 1.0
    g = -jnp.log(-jnp.log(u + 1e-8) + 1e-8)
    w = (x_ref[...] + g) / TEMP
    w = jnp.where(jj < COLS, w, -jnp.inf)

    m = jnp.max(w, axis=1, keepdims=True)
    idxb = jnp.min(
        jnp.where(w == m, jj, jnp.int32(2**31 - 1)), axis=1, keepdims=True
    )

    @pl.when(cb == 0)
    def _():
        val_ref[...] = m
        idx_ref[...] = idxb

    @pl.when(cb > 0)
    def _():
        better = m > val_ref[...]
        val_ref[...] = jnp.where(better, m, val_ref[...])
        idx_ref[...] = jnp.where(better, idxb, idx_ref[...])

    @pl.when(cb == NCB - 1)
    def _():
        def _zdrain(k, c):
            _zero_copy(k, zbuf_ref, out_ref, zsem).wait()
            return c

        lax.fori_loop(0, NCB, _zdrain, 0)

        icopy = pltpu.make_async_copy(idx_ref, idxs_ref, ssem)
        icopy.start()
        icopy.wait()

        idxg = jnp.reshape(idx_ref[...], (_NG, 8))
        base = (idxg // 128) * 128
        ll = lax.broadcasted_iota(jnp.int32, (_NG, 8, 8, 128), 3)
        tgt = base[:, :, None, None] + ll
        want = idxg[:, None, :, None]
        patch_ref[...] = (tgt == want).astype(jnp.float32)

        def _scopy(i):
            gidx = i // 8
            d = i - gidx * 8
            c = idxs_ref[i, 0]
            b = pl.multiple_of((c // 128) * 128, 128)
            row0 = pl.multiple_of(gidx * 8, 8)
            return pltpu.make_async_copy(
                patch_ref.at[gidx, d],
                out_ref.at[pl.ds(row0, 8), pl.ds(b, 128)],
                ssem,
            )

        def _sstart(i, c):
            _scopy(i).start()
            return c

        def _sdrain(i, c):
            _scopy(i).wait()
            return c

        lax.fori_loop(0, ROWS, _sstart, 0)
        lax.fori_loop(0, ROWS, _sdrain, 0)


@jax.jit
def kernel(logits):
    return pl.pallas_call(
        _fused_kernel,
        grid=(NCB,),
        in_specs=[pl.BlockSpec((ROWS, BC), lambda cb: (0, cb))],
        out_specs=pl.BlockSpec(memory_space=pl.ANY),
        out_shape=jax.ShapeDtypeStruct((ROWS, COLS), jnp.float32),
        scratch_shapes=[
            pltpu.VMEM((ROWS, 1), jnp.float32),
            pltpu.VMEM((ROWS, 1), jnp.int32),
            pltpu.VMEM((ROWS, BC), jnp.float32),
            pltpu.VMEM((_NG, 8, 8, 128), jnp.float32),
            pltpu.SMEM((ROWS, 1), jnp.int32),
            pltpu.SemaphoreType.DMA,
            pltpu.SemaphoreType.DMA,
        ],
    )(logits)
